# baseline (speedup 1.0000x reference)
import functools

import jax
import jax.numpy as jnp
from jax import lax
from jax.experimental import pallas as pl
from jax.experimental.pallas import tpu as pltpu
from jax.experimental.pallas import tpu_sc as plsc

C = 192
H = 384
W = 384
HW = H * W
K = 8192
L = 16
NC, NS = 2, 16
NW = NC * NS
CHUNK = HW // NW
PW = 256
NP = CHUNK // PW
CV = C // L
PV = PW // L
KV = K // L
VP = 256
RB = 32
NE = 48


def _sc_body(x_hbm, mask_hbm, flat_hbm, vals_hbm, out_hbm,
             xb, maskb, flatb, wloc, wkid, sloc, skid, ends,
             rows0, rows1, cur, gsem0, gsem1):
    rowss, gsms = (rows0, rows1), (gsem0, gsem1)
    wid = lax.axis_index("s") * NC + lax.axis_index("c")
    base = wid * CHUNK

    pltpu.sync_copy(mask_hbm.at[pl.ds(base, CHUNK)], maskb)
    pltpu.sync_copy(flat_hbm, flatb)

    iota = lax.iota(jnp.int32, L)
    zero16 = jnp.zeros((L,), jnp.int32)
    lane0 = iota == 0

    def wfilt(i, nk):
        v = flatb[pl.ds(i * L, L)]
        loc = v - base
        m = (loc >= 0) & (loc < CHUNK)
        cs = plsc.cumsum(m.astype(jnp.int32))
        pos = nk + cs - 1
        plsc.store_scatter(wloc, [pos], loc, mask=m)
        plsc.store_scatter(wkid, [pos], iota + i * L, mask=m)
        return nk + cs[L - 1]

    nk = lax.fori_loop(0, KV, wfilt, jnp.int32(0))
    nkv = (nk + (L - 1)) // L

    def sfil_piece(p, cum):
        pbase = p * PW

        def pfilt(i, np_):
            lv = wloc[pl.ds(i * L, L)]
            kv = wkid[pl.ds(i * L, L)]
            m = ((iota + i * L) < nk) & (lv >= pbase) & (lv < pbase + PW)
            cs = plsc.cumsum(m.astype(jnp.int32))
            pos = np_ + cs - 1
            plsc.store_scatter(sloc, [pos], lv, mask=m)
            plsc.store_scatter(skid, [pos], kv, mask=m)
            return np_ + cs[L - 1]

        cum2 = lax.fori_loop(0, nkv, pfilt, cum)
        plsc.store_scatter(ends, [jnp.full((L,), p, jnp.int32)],
                           jnp.full((L,), cum2, jnp.int32), mask=lane0)
        return cum2

    lax.fori_loop(0, NP, sfil_piece, jnp.int32(0))
    for q in range(2 * RB // L):
        skid[pl.ds(nk + q * L, L)] = zero16

    nbatw = (nk + RB - 1) // RB

    def gather(b, s):
        return pltpu.make_async_copy(
            vals_hbm.at[skid.at[pl.ds(b * RB, RB)]], rowss[s], gsms[s])

    @pl.when(nbatw > 0)
    def _g0():
        gather(0, 0).start()

    @pl.when(nbatw > 1)
    def _g1():
        gather(1, 1).start()

    def multiply(p):
        pbase = p * PW
        mvs = [maskb[pl.ds(pbase + v * L, L)] for v in range(PV)]

        @plsc.parallel_loop(0, C, unroll=8)
        def _mulc(c):
            for v in range(PV):
                xb[c, pl.ds(v * L, L)] = xb[c, pl.ds(v * L, L)] * mvs[v]

    def do_piece(p, lastw):
        pbase = p * PW
        pltpu.sync_copy(x_hbm.at[:, pl.ds(base + pbase, PW)], xb)
        multiply(p)

        endv = ends[pl.ds(p, L)]
        end_p = endv[0]
        prevv = ends[pl.ds(jnp.maximum(p - 1, 0), L)]
        start_p = jnp.where(p > 0, prevv[0], 0)

        bs = start_p // RB
        be = (end_p + RB - 1) // RB

        def batch_body(b, lw):
            @pl.when(b > lw)
            def _adv():
                for s in range(2):
                    @pl.when((b & 1) == s)
                    def _wait_copy():
                        gather(b, s).wait()

                        def cpy(r, _r):
                            for v in range(VP // L):
                                cur[r, pl.ds(v * L, L)] = (
                                    rowss[s][r, pl.ds(v * L, L)])
                            return _r

                        lax.fori_loop(0, RB, cpy, 0)

                        @pl.when(b + 2 < nbatw)
                        def _refire():
                            gather(b + 2, s).start()

            lw2 = jnp.maximum(lw, b)

            for q in range(RB // L):
                slv = sloc[pl.ds(b * RB + q * L, L)]
                for j in range(L):
                    o = b * RB + q * L + j
                    valid = jnp.full((L,), (o >= start_p) & (o < end_p))
                    ocol = jnp.full((L,), slv[j], jnp.int32) - pbase
                    for t in range(CV):
                        plsc.store_scatter(
                            xb, [iota + t * L, ocol],
                            cur[q * L + j, pl.ds(t * L, L)], mask=valid)
            return lw2

        lastw = lax.fori_loop(bs, be, batch_body, lastw)

        pltpu.sync_copy(xb, out_hbm.at[:, pl.ds(base + pbase, PW)])
        return lastw

    lax.fori_loop(0, NP, do_piece, jnp.int32(-1))


@jax.jit
def kernel(input_x, mask, idx, vals):
    x2 = input_x.reshape(C, HW)
    mask_f = mask.astype(input_x.dtype).reshape(HW)
    flat = (idx[:, 0] * W + idx[:, 1]).astype(jnp.int32)
    vals_p = jnp.pad(vals, ((0, 0), (0, VP - C)))

    mesh = plsc.VectorSubcoreMesh(core_axis_name="c", subcore_axis_name="s")
    run = functools.partial(
        pl.kernel,
        out_type=jax.ShapeDtypeStruct((C, HW), jnp.float32),
        mesh=mesh,
        scratch_types=[
            pltpu.VMEM((C, PW), jnp.float32),
            pltpu.VMEM((CHUNK,), jnp.float32),
            pltpu.VMEM((K,), jnp.int32),
            pltpu.VMEM((K,), jnp.int32),
            pltpu.VMEM((K,), jnp.int32),
            pltpu.VMEM((K,), jnp.int32),
            pltpu.VMEM((K + 2 * RB,), jnp.int32),
            pltpu.VMEM((NE,), jnp.int32),
            pltpu.VMEM((RB, VP), jnp.float32),
            pltpu.VMEM((RB, VP), jnp.float32),
            pltpu.VMEM((RB, VP), jnp.float32),
            pltpu.SemaphoreType.DMA,
            pltpu.SemaphoreType.DMA,
        ],
        compiler_params=pltpu.CompilerParams(needs_layout_passes=False),
    )(_sc_body)
    out = run(x2, mask_f, flat, vals_p)
    return out.reshape(1, C, H, W)
